# baseline (speedup 1.0000x reference)
import jax
import jax.numpy as jnp
from jax import lax
from jax.experimental import pallas as pl
from jax.experimental.pallas import tpu as pltpu
from jax.experimental.pallas import tpu_sc as plsc

N = 10000
E = 320000
D_IN = 128
H = 256
C = 230
G = 64

NC = 2
NS = 16
L = 16
NW = NC * NS
RW = 320
NPAD = NW * RW
EMAX = 16384
CH = 64
CHS = 128
SCH = 2000
CR = 16
HV = H // L
BR = 512

_SC_PARAMS = pltpu.CompilerParams(needs_layout_passes=False)


LCAP = 1024


def _compact_body(src_hbm, dst_hbm, deg_hbm, ssrc_hbm, offs_hbm,
                  sbuf0, sbuf1, dbuf0, dbuf1, lsrc, ldst, csrc, cdst,
                  ssrc, degacc, offbuf, curoff, sems, semd):
    w = lax.axis_index("c") * NS + lax.axis_index("s")
    lo = w * RW
    lanes = lax.iota(jnp.int32, L)
    zer = jnp.zeros((L,), jnp.float32)

    def zb(i, _):
        degacc[pl.ds(i * L, L)] = zer
        return 0
    lax.fori_loop(0, (RW + L) // L, zb, 0)

    lanebase = lanes * LCAP
    NSC = E // SCH
    sbufs = (sbuf0, sbuf1)
    dbufs = (dbuf0, dbuf1)

    def start(ci, b):
        pltpu.async_copy(src_hbm.at[pl.ds(ci * SCH, SCH)], sbufs[b], sems)
        pltpu.async_copy(dst_hbm.at[pl.ds(ci * SCH, SCH)], dbufs[b], semd)

    def wait(b):
        pltpu.make_async_copy(src_hbm.at[pl.ds(0, SCH)],
                              sbufs[b], sems).wait()
        pltpu.make_async_copy(dst_hbm.at[pl.ds(0, SCH)],
                              dbufs[b], semd).wait()

    def scan(b, ptrv):
        def vec(i, ptrv):
            d16 = dbufs[b][pl.ds(i * L, L)]
            s16 = sbufs[b][pl.ds(i * L, L)]
            dl = d16 - lo
            m = (dl >= 0) & (dl < RW)
            pos = lanebase + jnp.minimum(ptrv, LCAP - 1)
            plsc.store_scatter(ldst, [pos], dl, mask=m)
            plsc.store_scatter(lsrc, [pos], s16, mask=m)
            return ptrv + m.astype(jnp.int32)
        return lax.fori_loop(0, SCH // L, vec, ptrv)

    start(0, 0)

    def chunk(cp, ptrv):
        start(2 * cp + 1, 1)
        wait(0)
        ptrv = scan(0, ptrv)

        @pl.when(cp + 1 < NSC // 2)
        def _():
            start(2 * cp + 2, 0)
        wait(1)
        return scan(1, ptrv)
    ptrv = lax.fori_loop(0, NSC // 2, chunk, jnp.zeros((L,), jnp.int32))

    def merge(l, ptr):
        cl = jnp.minimum(ptrv[l], LCAP)
        nv = (cl + L - 1) // L

        def mv(i, _):
            cdst[pl.ds(ptr + i * L, L)] = ldst[pl.ds(l * LCAP + i * L, L)]
            csrc[pl.ds(ptr + i * L, L)] = lsrc[pl.ds(l * LCAP + i * L, L)]
            return 0
        lax.fori_loop(0, nv, mv, 0)
        return jnp.minimum(ptr + cl, EMAX - 2 * CHS - L)
    ptr = jnp.int32(0)
    for l in range(L):
        ptr = merge(l, ptr)

    dum_d = jnp.full((L,), RW, jnp.int32)
    dum_s = jnp.zeros((L,), jnp.int32)
    for k in range(2 * CHS // L):
        cdst[pl.ds(ptr + k * L, L)] = dum_d
        csrc[pl.ds(ptr + k * L, L)] = dum_s

    ones = jnp.ones((L,), jnp.float32)
    nvec = (ptr + L - 1) // L

    def degb(i, _):
        dl = cdst[pl.ds(i * L, L)]
        for l in range(L):
            plsc.addupdate_scatter(degacc, [dl], ones, mask=lanes == l)
        return 0
    lax.fori_loop(0, nvec, degb, 0)

    carry = jnp.int32(0)
    for v in range(RW // L):
        c16 = degacc[pl.ds(v * L, L)].astype(jnp.int32)
        incl = plsc.cumsum(c16)
        offbuf[pl.ds(v * L, L)] = incl - c16 + carry
        carry = carry + jnp.max(incl)
    offbuf[pl.ds(RW, L)] = jnp.full((L,), carry, jnp.int32)

    def cpb(i, _):
        curoff[pl.ds(i * L, L)] = offbuf[pl.ds(i * L, L)]
        return 0
    lax.fori_loop(0, (RW + L) // L, cpb, 0)

    def place(i, _):
        dl = cdst[pl.ds(i * L, L)]
        s16 = csrc[pl.ds(i * L, L)]
        for l in range(L):
            ml = lanes == l
            pos = plsc.load_gather(curoff, [dl], mask=ml)
            plsc.store_scatter(ssrc, [pos], s16, mask=ml)
            plsc.store_scatter(curoff, [dl], pos + 1, mask=ml)
        return 0
    lax.fori_loop(0, nvec, place, 0)

    zs = jnp.zeros((L,), jnp.int32)
    for k in range(2 * CHS // L):
        ssrc[pl.ds(carry + k * L, L)] = zs

    pltpu.sync_copy(degacc.at[pl.ds(0, RW)], deg_hbm.at[pl.ds(lo, RW)])
    pltpu.sync_copy(ssrc, ssrc_hbm.at[w])
    pltpu.sync_copy(offbuf, offs_hbm.at[w])


def _scatter_body(hp_hbm, ssrc_hbm, offs_hbm, s_hbm,
                  idxv0, idxv1, rows0, rows1, offbuf, acc,
                  gsem0, gsem1, gsem2, gsem3, isem0, isem1):
    w = lax.axis_index("c") * NS + lax.axis_index("s")
    pltpu.sync_copy(offs_hbm.at[w], offbuf)
    total = offbuf[pl.ds(RW, L)][0]
    nch = (total + CHS - 1) // CHS
    nch = jnp.maximum(nch + lax.rem(nch, 2), 2)
    zer = jnp.zeros((L,), jnp.float32)

    def zb(i, _):
        acc[pl.ds(i * L, L)] = zer
        return 0
    lax.fori_loop(0, RW * HV, zb, 0)

    idxvs = (idxv0, idxv1)
    rowss = (rows0, rows1)
    isems = (isem0, isem1)
    gsems = ((gsem0, gsem2), (gsem1, gsem3))

    def start_idx(ci, b):
        pltpu.async_copy(ssrc_hbm.at[w, pl.ds(ci * CHS, CHS)],
                         idxvs[b], isems[b])

    def wait_idx(b):
        pltpu.make_async_copy(ssrc_hbm.at[w, pl.ds(0, CHS)],
                              idxvs[b], isems[b]).wait()

    HCH = CHS // 2

    def start_gather(b):
        pltpu.async_copy(hp_hbm.at[idxvs[b].at[pl.ds(0, HCH)]],
                         rowss[b].at[pl.ds(0, HCH)], gsems[b][0])
        pltpu.async_copy(hp_hbm.at[idxvs[b].at[pl.ds(HCH, HCH)]],
                         rowss[b].at[pl.ds(HCH, HCH)], gsems[b][1])

    def wait_gather(b):
        pltpu.make_async_copy(hp_hbm.at[pl.ds(0, HCH)],
                              rowss[b].at[pl.ds(0, HCH)], gsems[b][0]).wait()
        pltpu.make_async_copy(hp_hbm.at[pl.ds(0, HCH)],
                              rowss[b].at[pl.ds(HCH, HCH)], gsems[b][1]).wait()

    def accum(b, base, r):
        rowsb = rowss[b]
        hi_chunk = jnp.minimum(base + CHS, total)

        def cond(st):
            r, e = st
            return (r < RW) & (e < hi_chunk)

        def body(st):
            r, e = st
            off_r = offbuf[pl.ds(r, L)][0]
            off_r1 = offbuf[pl.ds(r + 1, L)][0]
            hi_e = jnp.minimum(off_r1, hi_chunk)
            cmask = jnp.full((L,), e > off_r)
            regs = tuple(
                jnp.where(cmask, acc[pl.ds(r * H + k * L, L)], 0.0)
                for k in range(HV))

            def ebody(e, regs):
                eb = e - base
                out = []
                for k in range(H // 32):
                    v32i = rowsb[eb, pl.ds(k * L, L)]
                    v32 = plsc.bitcast(v32i, jnp.bfloat16)
                    va, vb = plsc.unpack(
                        v32, format=plsc.PackFormat.INTERLEAVED,
                        preferred_element_type=jnp.float32)
                    out.append(regs[2 * k] + va)
                    out.append(regs[2 * k + 1] + vb)
                return tuple(out)
            regs = lax.fori_loop(e, hi_e, ebody, regs)
            for k in range(HV):
                acc[pl.ds(r * H + k * L, L)] = regs[k]
            r = jnp.where(off_r1 <= hi_chunk, r + 1, r)
            return (r, hi_e)
        r, _ = lax.while_loop(cond, body, (r, base))
        return r

    start_idx(0, 0)
    start_idx(1, 1)
    wait_idx(0)
    start_gather(0)
    nchp = nch // 2

    def cb(cp, r):
        wait_idx(1)
        start_gather(1)
        wait_gather(0)
        r = accum(0, 2 * cp * CHS, r)

        @pl.when(cp + 1 < nchp)
        def _():
            start_idx(2 * cp + 2, 0)
            wait_idx(0)
            start_gather(0)
        wait_gather(1)
        r = accum(1, (2 * cp + 1) * CHS, r)

        @pl.when(cp + 1 < nchp)
        def _():
            start_idx(2 * cp + 3, 1)
        return r
    lax.fori_loop(0, nchp, cb, jnp.int32(0))
    pltpu.sync_copy(acc.at[pl.ds(0, RW * H)],
                    s_hbm.at[pl.ds(w * RW * H, RW * H)])


def _pool_body(y_hbm, batch_hbm, z_hbm, bbuf, ybuf, zbuf, pacc):
    w = lax.axis_index("c") * NS + lax.axis_index("s")
    g0 = 2 * w
    zer = jnp.zeros((L,), jnp.float32)
    neg = jnp.full((L,), -3.0e38, jnp.float32)

    def cb(ci, carry):
        pltpu.sync_copy(batch_hbm.at[pl.ds(ci * SCH, SCH)], bbuf)

        def vb(i, carry):
            lo, mid, hi = carry
            b16 = bbuf[pl.ds(i * L, L)]
            lo = lo + jnp.max(plsc.all_reduce_population_count(b16 < g0))
            mid = mid + jnp.max(plsc.all_reduce_population_count(b16 < g0 + 1))
            hi = hi + jnp.max(plsc.all_reduce_population_count(b16 < g0 + 2))
            return (lo, mid, hi)
        return lax.fori_loop(0, SCH // L, vb, carry)
    z0 = jnp.int32(0)
    lo, mid, hi = lax.fori_loop(0, NPAD // SCH, cb, (z0, z0, z0))

    for k in range(HV):
        pacc[pl.ds(k * L, L)] = zer
        pacc[pl.ds(H + k * L, L)] = neg
        pacc[pl.ds(2 * H + k * L, L)] = zer
        pacc[pl.ds(3 * H + k * L, L)] = neg

    nchr = (hi - lo + CR - 1) // CR

    def rb(ci, _):
        base = lo + ci * CR
        pltpu.sync_copy(y_hbm.at[pl.ds(base * H, CR * H)], ybuf)

        def rr(j, _):
            r = base + j

            @pl.when(r < hi)
            def _():
                off = jnp.where(r < mid, 0, 2 * H)
                for k in range(HV):
                    v = ybuf[pl.ds(j * H + k * L, L)]
                    plsc.addupdate(pacc.at[pl.ds(off + k * L, L)], v)
                    cur = pacc[pl.ds(off + H + k * L, L)]
                    pacc[pl.ds(off + H + k * L, L)] = jnp.maximum(cur, v)
            return 0
        lax.fori_loop(0, CR, rr, 0)
        return 0
    lax.fori_loop(0, nchr, rb, 0)

    c0v = jnp.full((L,), mid - lo, jnp.int32)
    c1v = jnp.full((L,), hi - mid, jnp.int32)
    inv0 = 1.0 / jnp.maximum(c0v.astype(jnp.float32), 1.0)
    inv1 = 1.0 / jnp.maximum(c1v.astype(jnp.float32), 1.0)
    ok0 = c0v > 0
    ok1 = c1v > 0
    for k in range(HV):
        zbuf[pl.ds(k * L, L)] = pacc[pl.ds(k * L, L)] * inv0
        zbuf[pl.ds(H + k * L, L)] = jnp.where(
            ok0, pacc[pl.ds(H + k * L, L)], 0.0)
        zbuf[pl.ds(2 * H + k * L, L)] = pacc[pl.ds(2 * H + k * L, L)] * inv1
        zbuf[pl.ds(3 * H + k * L, L)] = jnp.where(
            ok1, pacc[pl.ds(3 * H + k * L, L)], 0.0)
    pltpu.sync_copy(zbuf, z_hbm.at[pl.ds(g0 * 2 * H, 4 * H)])


def _pack_words(lo, hi):
    lob = lax.bitcast_convert_type(lo.astype(jnp.bfloat16),
                                   jnp.int16).astype(jnp.int32) & 0xFFFF
    hib = lax.bitcast_convert_type(hi.astype(jnp.bfloat16),
                                   jnp.int16).astype(jnp.int32)
    return lob | lax.shift_left(hib, 16)


def _t10_body(x_ref, deg_ref, w_ref, wlo_ref, whi_ref, o_ref, ow_ref):
    dinv = lax.rsqrt(deg_ref[...] + 1.0)
    x = x_ref[...]
    o_ref[...] = dinv * jnp.dot(x, w_ref[...],
                                preferred_element_type=jnp.float32)
    lo = dinv * jnp.dot(x, wlo_ref[...], preferred_element_type=jnp.float32)
    hi = dinv * jnp.dot(x, whi_ref[...], preferred_element_type=jnp.float32)
    ow_ref[...] = _pack_words(lo, hi)


def _t1i_body(a_ref, st_ref, g_ref, be_ref, deg_ref, w_ref, wlo_ref,
              whi_ref, o_ref, ow_ref):
    s = st_ref[...]
    mean = s[0:1, :] * (1.0 / N)
    var = s[1:2, :] * (1.0 / N) - mean * mean
    rstd = lax.rsqrt(var + 1e-5)
    y = jnp.maximum((a_ref[...] - mean) * rstd * g_ref[...] + be_ref[...], 0.0)
    dinv = lax.rsqrt(deg_ref[...] + 1.0)
    o_ref[...] = dinv * jnp.dot(y, w_ref[...],
                                preferred_element_type=jnp.float32)
    lo = dinv * jnp.dot(y, wlo_ref[...], preferred_element_type=jnp.float32)
    hi = dinv * jnp.dot(y, whi_ref[...], preferred_element_type=jnp.float32)
    ow_ref[...] = _pack_words(lo, hi)


def _t3_body(s_ref, hp_ref, deg_ref, b_ref, a_ref, st_ref):
    i = pl.program_id(0)
    dinv = lax.rsqrt(deg_ref[...] + 1.0)
    a = dinv * (s_ref[...] + hp_ref[...]) + b_ref[...]
    a_ref[...] = a
    rows = i * BR + lax.broadcasted_iota(jnp.int32, (BR, 1), 0)
    am = jnp.where(rows < N, a, 0.0)
    ps = jnp.concatenate([jnp.sum(am, 0, keepdims=True),
                          jnp.sum(am * am, 0, keepdims=True)], 0)

    @pl.when(i == 0)
    def _():
        st_ref[...] = jnp.zeros_like(st_ref)
    st_ref[...] += ps


def _t4a_body(a_ref, st_ref, g_ref, be_ref, y_ref):
    s = st_ref[...]
    mean = s[0:1, :] * (1.0 / N)
    var = s[1:2, :] * (1.0 / N) - mean * mean
    rstd = lax.rsqrt(var + 1e-5)
    y_ref[...] = jnp.maximum(
        (a_ref[...] - mean) * rstd * g_ref[...] + be_ref[...], 0.0)


def _clf_body(z_ref, w0_ref, b0_ref, w1_ref, b1_ref, w2_ref, b2_ref, o_ref):
    z = jnp.maximum(jnp.dot(z_ref[...], w0_ref[...],
                            preferred_element_type=jnp.float32) + b0_ref[...],
                    0.0)
    z = jnp.maximum(jnp.dot(z, w1_ref[...],
                            preferred_element_type=jnp.float32) + b1_ref[...],
                    0.0)
    lg = jnp.dot(z, w2_ref[...],
                 preferred_element_type=jnp.float32) + b2_ref[...]
    mx = jnp.max(lg, axis=1, keepdims=True)
    sh = lg - mx
    lse = jnp.log(jnp.sum(jnp.exp(sh), axis=1, keepdims=True))
    o_ref[...] = sh - lse


def _row_specs(feat):
    return pl.BlockSpec((BR, feat), lambda i: (i, 0))


def _const_spec(shape):
    return pl.BlockSpec(shape, lambda i: (0, 0))


def _t10(xp, deg_col, W0, Wlo, Whi):
    return pl.pallas_call(
        _t10_body,
        grid=(NPAD // BR,),
        in_specs=[_row_specs(D_IN), _row_specs(1), _const_spec((D_IN, H)),
                  _const_spec((D_IN, H // 2)), _const_spec((D_IN, H // 2))],
        out_specs=[_row_specs(H), _row_specs(H // 2)],
        out_shape=[jax.ShapeDtypeStruct((NPAD, H), jnp.float32),
                   jax.ShapeDtypeStruct((NPAD, H // 2), jnp.int32)],
    )(xp, deg_col, W0, Wlo, Whi)


def _t1i(a, st, g, be, deg_col, W, Wlo, Whi):
    return pl.pallas_call(
        _t1i_body,
        grid=(NPAD // BR,),
        in_specs=[_row_specs(H), _const_spec((2, H)), _const_spec((1, H)),
                  _const_spec((1, H)), _row_specs(1), _const_spec((H, H)),
                  _const_spec((H, H // 2)), _const_spec((H, H // 2))],
        out_specs=[_row_specs(H), _row_specs(H // 2)],
        out_shape=[jax.ShapeDtypeStruct((NPAD, H), jnp.float32),
                   jax.ShapeDtypeStruct((NPAD, H // 2), jnp.int32)],
    )(a, st, g, be, deg_col, W, Wlo, Whi)


def _t3(S, hp, deg_col, b):
    return pl.pallas_call(
        _t3_body,
        grid=(NPAD // BR,),
        in_specs=[_row_specs(H), _row_specs(H), _row_specs(1),
                  _const_spec((1, H))],
        out_specs=[_row_specs(H), _const_spec((2, H))],
        out_shape=[jax.ShapeDtypeStruct((NPAD, H), jnp.float32),
                   jax.ShapeDtypeStruct((2, H), jnp.float32)],
    )(S, hp, deg_col, b)


def _t4a(a, st, g, be):
    return pl.pallas_call(
        _t4a_body,
        grid=(NPAD // BR,),
        in_specs=[_row_specs(H), _const_spec((2, H)), _const_spec((1, H)),
                  _const_spec((1, H))],
        out_specs=_row_specs(H),
        out_shape=jax.ShapeDtypeStruct((NPAD, H), jnp.float32),
    )(a, st, g, be)


def _clf(z, cW0, cb0, cW1, cb1, cW2, cb2):
    return pl.pallas_call(
        _clf_body,
        out_shape=jax.ShapeDtypeStruct((G, C), jnp.float32),
    )(z, cW0, cb0, cW1, cb1, cW2, cb2)


def kernel(x, edge_index, batch, W0, b0, g0, be0, W1, b1, g1, be1,
           W2, b2, g2, be2, cW0, cb0, cW1, cb1, cW2, cb2):
    src = edge_index[0]
    dst = edge_index[1]
    xp = jnp.pad(x, ((0, NPAD - N), (0, 0)))
    batch_p = jnp.pad(batch, (0, NPAD - N), constant_values=127)
    mesh = plsc.VectorSubcoreMesh(core_axis_name="c", subcore_axis_name="s",
                                  num_cores=NC, num_subcores=NS)

    deg, ssrc, offs = pl.kernel(
        _compact_body,
        out_type=(jax.ShapeDtypeStruct((NPAD,), jnp.float32),
                  jax.ShapeDtypeStruct((NW, EMAX), jnp.int32),
                  jax.ShapeDtypeStruct((NW, RW + L), jnp.int32)),
        mesh=mesh,
        compiler_params=_SC_PARAMS,
        scratch_types=[pltpu.VMEM((SCH,), jnp.int32),
                       pltpu.VMEM((SCH,), jnp.int32),
                       pltpu.VMEM((SCH,), jnp.int32),
                       pltpu.VMEM((SCH,), jnp.int32),
                       pltpu.VMEM((L * LCAP,), jnp.int32),
                       pltpu.VMEM((L * LCAP,), jnp.int32),
                       pltpu.VMEM((EMAX,), jnp.int32),
                       pltpu.VMEM((EMAX,), jnp.int32),
                       pltpu.VMEM((EMAX,), jnp.int32),
                       pltpu.VMEM((RW + L,), jnp.float32),
                       pltpu.VMEM((RW + L,), jnp.int32),
                       pltpu.VMEM((RW + L,), jnp.int32),
                       pltpu.SemaphoreType.DMA,
                       pltpu.SemaphoreType.DMA],
    )(src, dst)
    deg_col = deg.reshape(NPAD, 1)

    def _scatter(hp):
        s_flat = pl.kernel(
            _scatter_body,
            out_type=jax.ShapeDtypeStruct((NPAD * H,), jnp.float32),
            mesh=mesh,
            compiler_params=_SC_PARAMS,
            scratch_types=[pltpu.VMEM((CHS,), jnp.int32),
                           pltpu.VMEM((CHS,), jnp.int32),
                           pltpu.VMEM((CHS, H // 2), jnp.int32),
                           pltpu.VMEM((CHS, H // 2), jnp.int32),
                           pltpu.VMEM((RW + L,), jnp.int32),
                           pltpu.VMEM((RW * H,), jnp.float32),
                           pltpu.SemaphoreType.DMA,
                           pltpu.SemaphoreType.DMA,
                           pltpu.SemaphoreType.DMA,
                           pltpu.SemaphoreType.DMA,
                           pltpu.SemaphoreType.DMA,
                           pltpu.SemaphoreType.DMA],
        )(hp, ssrc, offs)
        return s_flat.reshape(NPAD, H)

    locols = jnp.asarray([32 * (m // 16) + m % 16 for m in range(H // 2)],
                         dtype=jnp.int32)
    hicols = locols + 16

    def _wsplit(W):
        return jnp.take(W, locols, axis=1), jnp.take(W, hicols, axis=1)

    hp, hpw = _t10(xp, deg_col, W0, *_wsplit(W0))
    layer = [(b0, g0, be0, W1), (b1, g1, be1, W2), (b2, g2, be2, None)]
    a = st = None
    for li, (b, g, be, Wn) in enumerate(layer):
        S = _scatter(hpw)
        a, st = _t3(S, hp, deg_col, b.reshape(1, H))
        if Wn is not None:
            hp, hpw = _t1i(a, st, g.reshape(1, H), be.reshape(1, H),
                           deg_col, Wn, *_wsplit(Wn))

    y = _t4a(a, st, g2.reshape(1, H), be2.reshape(1, H))

    z = pl.kernel(
        _pool_body,
        out_type=jax.ShapeDtypeStruct((G * 2 * H,), jnp.float32),
        mesh=mesh,
        compiler_params=_SC_PARAMS,
        scratch_types=[pltpu.VMEM((SCH,), jnp.int32),
                       pltpu.VMEM((CR * H,), jnp.float32),
                       pltpu.VMEM((4 * H,), jnp.float32),
                       pltpu.VMEM((4 * H,), jnp.float32)],
    )(y.reshape(NPAD * H), batch_p).reshape(G, 2 * H)

    return _clf(z, cW0, cb0.reshape(1, H), cW1, cb1.reshape(1, H // 2),
                cW2, cb2.reshape(1, C))


# speedup vs baseline: 1.0120x; 1.0120x over previous
import jax
import jax.numpy as jnp
from jax import lax
from jax.experimental import pallas as pl
from jax.experimental.pallas import tpu as pltpu
from jax.experimental.pallas import tpu_sc as plsc

N = 10000
E = 320000
D_IN = 128
H = 256
C = 230
G = 64

NC = 2
NS = 16
L = 16
NW = NC * NS
RW = 320
NPAD = NW * RW
EMAX = 16384
CH = 64
CHS = 128
SCH = 2000
CR = 16
HV = H // L
BR = 512

_SC_PARAMS = pltpu.CompilerParams(needs_layout_passes=False)


LCAP = 1024


def _compact_body(src_hbm, dst_hbm, deg_hbm, ssrc_hbm, offs_hbm,
                  sbuf0, sbuf1, dbuf0, dbuf1, lsrc, ldst, csrc, cdst,
                  ssrc, degacc, offbuf, curoff, sems, semd):
    w = lax.axis_index("c") * NS + lax.axis_index("s")
    lo = w * RW
    lanes = lax.iota(jnp.int32, L)
    zer = jnp.zeros((L,), jnp.float32)

    def zb(i, _):
        degacc[pl.ds(i * L, L)] = zer
        return 0
    lax.fori_loop(0, (RW + L) // L, zb, 0)

    lanebase = lanes * LCAP
    NSC = E // SCH
    sbufs = (sbuf0, sbuf1)
    dbufs = (dbuf0, dbuf1)

    def start(ci, b):
        pltpu.async_copy(src_hbm.at[pl.ds(ci * SCH, SCH)], sbufs[b], sems)
        pltpu.async_copy(dst_hbm.at[pl.ds(ci * SCH, SCH)], dbufs[b], semd)

    def wait(b):
        pltpu.make_async_copy(src_hbm.at[pl.ds(0, SCH)],
                              sbufs[b], sems).wait()
        pltpu.make_async_copy(dst_hbm.at[pl.ds(0, SCH)],
                              dbufs[b], semd).wait()

    def scan(b, ptrv):
        def vec(i, ptrv):
            d16 = dbufs[b][pl.ds(i * L, L)]
            s16 = sbufs[b][pl.ds(i * L, L)]
            dl = d16 - lo
            m = (dl >= 0) & (dl < RW)
            pos = lanebase + jnp.minimum(ptrv, LCAP - 1)
            plsc.store_scatter(ldst, [pos], dl, mask=m)
            plsc.store_scatter(lsrc, [pos], s16, mask=m)
            return ptrv + m.astype(jnp.int32)
        return lax.fori_loop(0, SCH // L, vec, ptrv)

    start(0, 0)

    def chunk(cp, ptrv):
        start(2 * cp + 1, 1)
        wait(0)
        ptrv = scan(0, ptrv)

        @pl.when(cp + 1 < NSC // 2)
        def _():
            start(2 * cp + 2, 0)
        wait(1)
        return scan(1, ptrv)
    ptrv = lax.fori_loop(0, NSC // 2, chunk, jnp.zeros((L,), jnp.int32))

    def merge(l, ptr):
        cl = jnp.minimum(ptrv[l], LCAP)
        nv = (cl + L - 1) // L

        def mv(i, _):
            cdst[pl.ds(ptr + i * L, L)] = ldst[pl.ds(l * LCAP + i * L, L)]
            csrc[pl.ds(ptr + i * L, L)] = lsrc[pl.ds(l * LCAP + i * L, L)]
            return 0
        lax.fori_loop(0, nv, mv, 0)
        return jnp.minimum(ptr + cl, EMAX - 2 * CHS - L)
    ptr = jnp.int32(0)
    for l in range(L):
        ptr = merge(l, ptr)

    dum_d = jnp.full((L,), RW, jnp.int32)
    dum_s = jnp.zeros((L,), jnp.int32)
    for k in range(2 * CHS // L):
        cdst[pl.ds(ptr + k * L, L)] = dum_d
        csrc[pl.ds(ptr + k * L, L)] = dum_s

    ones = jnp.ones((L,), jnp.float32)
    nvec = (ptr + L - 1) // L

    def degb(i, _):
        dl = cdst[pl.ds(i * L, L)]
        for l in range(L):
            plsc.addupdate_scatter(degacc, [dl], ones, mask=lanes == l)
        return 0
    lax.fori_loop(0, nvec, degb, 0)

    carry = jnp.int32(0)
    for v in range(RW // L):
        c16 = degacc[pl.ds(v * L, L)].astype(jnp.int32)
        incl = plsc.cumsum(c16)
        offbuf[pl.ds(v * L, L)] = incl - c16 + carry
        carry = carry + jnp.max(incl)
    offbuf[pl.ds(RW, L)] = jnp.full((L,), carry, jnp.int32)

    def cpb(i, _):
        curoff[pl.ds(i * L, L)] = offbuf[pl.ds(i * L, L)]
        return 0
    lax.fori_loop(0, (RW + L) // L, cpb, 0)

    def place(i, _):
        dl = cdst[pl.ds(i * L, L)]
        s16 = csrc[pl.ds(i * L, L)]
        for l in range(L):
            ml = lanes == l
            pos = plsc.load_gather(curoff, [dl], mask=ml)
            plsc.store_scatter(ssrc, [pos], s16, mask=ml)
            plsc.store_scatter(curoff, [dl], pos + 1, mask=ml)
        return 0
    lax.fori_loop(0, nvec, place, 0)

    zs = jnp.zeros((L,), jnp.int32)
    for k in range(2 * CHS // L):
        ssrc[pl.ds(carry + k * L, L)] = zs

    pltpu.sync_copy(degacc.at[pl.ds(0, RW)], deg_hbm.at[pl.ds(lo, RW)])
    pltpu.sync_copy(ssrc, ssrc_hbm.at[w])
    pltpu.sync_copy(offbuf, offs_hbm.at[w])


def _scatter_body(hp_hbm, ssrc_hbm, offs_hbm, s_hbm,
                  idxv0, idxv1, rows0, rows1, offbuf, acc,
                  gsem0, gsem1, isem0, isem1):
    w = lax.axis_index("c") * NS + lax.axis_index("s")
    pltpu.sync_copy(offs_hbm.at[w], offbuf)
    total = offbuf[pl.ds(RW, L)][0]
    nch = (total + CHS - 1) // CHS
    nch = jnp.maximum(nch + lax.rem(nch, 2), 2)
    zer = jnp.zeros((L,), jnp.float32)

    def zb(i, _):
        acc[pl.ds(i * L, L)] = zer
        return 0
    lax.fori_loop(0, RW * HV, zb, 0)

    idxvs = (idxv0, idxv1)
    rowss = (rows0, rows1)
    isems = (isem0, isem1)
    gsems = (gsem0, gsem1)

    def start_idx(ci, b):
        pltpu.async_copy(ssrc_hbm.at[w, pl.ds(ci * CHS, CHS)],
                         idxvs[b], isems[b])

    def wait_idx(b):
        pltpu.make_async_copy(ssrc_hbm.at[w, pl.ds(0, CHS)],
                              idxvs[b], isems[b]).wait()

    def start_gather(b):
        pltpu.async_copy(hp_hbm.at[idxvs[b]], rowss[b], gsems[b])

    def wait_gather(b):
        pltpu.make_async_copy(hp_hbm.at[pl.ds(0, CHS)],
                              rowss[b], gsems[b]).wait()

    def accum(b, base, r):
        rowsb = rowss[b]
        hi_chunk = jnp.minimum(base + CHS, total)

        def cond(st):
            r, e = st
            return (r < RW) & (e < hi_chunk)

        def body(st):
            r, e = st
            off_r = offbuf[pl.ds(r, L)][0]
            off_r1 = offbuf[pl.ds(r + 1, L)][0]
            hi_e = jnp.minimum(off_r1, hi_chunk)
            cmask = jnp.full((L,), e > off_r)
            regs = tuple(
                jnp.where(cmask, acc[pl.ds(r * H + k * L, L)], 0.0)
                for k in range(HV))

            def ebody(e, regs):
                eb = e - base
                out = []
                for k in range(H // 32):
                    v32i = rowsb[eb, pl.ds(k * L, L)]
                    v32 = plsc.bitcast(v32i, jnp.bfloat16)
                    va, vb = plsc.unpack(
                        v32, format=plsc.PackFormat.INTERLEAVED,
                        preferred_element_type=jnp.float32)
                    out.append(regs[2 * k] + va)
                    out.append(regs[2 * k + 1] + vb)
                return tuple(out)
            regs = lax.fori_loop(e, hi_e, ebody, regs)
            for k in range(HV):
                acc[pl.ds(r * H + k * L, L)] = regs[k]
            r = jnp.where(off_r1 <= hi_chunk, r + 1, r)
            return (r, hi_e)
        r, _ = lax.while_loop(cond, body, (r, base))
        return r

    start_idx(0, 0)
    start_idx(1, 1)
    wait_idx(0)
    start_gather(0)
    nchp = nch // 2

    def cb(cp, r):
        wait_idx(1)
        start_gather(1)
        wait_gather(0)
        r = accum(0, 2 * cp * CHS, r)

        @pl.when(cp + 1 < nchp)
        def _():
            start_idx(2 * cp + 2, 0)
            wait_idx(0)
            start_gather(0)
        wait_gather(1)
        r = accum(1, (2 * cp + 1) * CHS, r)

        @pl.when(cp + 1 < nchp)
        def _():
            start_idx(2 * cp + 3, 1)
        return r
    lax.fori_loop(0, nchp, cb, jnp.int32(0))
    pltpu.sync_copy(acc.at[pl.ds(0, RW * H)],
                    s_hbm.at[pl.ds(w * RW * H, RW * H)])


def _pool_body(y_hbm, batch_hbm, z_hbm, bbuf, ybuf0, ybuf1, zbuf, pacc,
               semy0, semy1):
    w = lax.axis_index("c") * NS + lax.axis_index("s")
    g0 = 2 * w
    zer = jnp.zeros((L,), jnp.float32)
    neg = jnp.full((L,), -3.0e38, jnp.float32)

    def cb(ci, carry):
        pltpu.sync_copy(batch_hbm.at[pl.ds(ci * SCH, SCH)], bbuf)

        def vb(i, carry):
            lo_v, mid_v, hi_v = carry
            b16 = bbuf[pl.ds(i * L, L)]
            lo_v = lo_v + (b16 < g0).astype(jnp.int32)
            mid_v = mid_v + (b16 < g0 + 1).astype(jnp.int32)
            hi_v = hi_v + (b16 < g0 + 2).astype(jnp.int32)
            return (lo_v, mid_v, hi_v)
        return lax.fori_loop(0, SCH // L, vb, carry)
    z0v = jnp.zeros((L,), jnp.int32)
    lo_v, mid_v, hi_v = lax.fori_loop(0, NPAD // SCH, cb, (z0v, z0v, z0v))
    lo = jnp.sum(lo_v)
    mid = jnp.sum(mid_v)
    hi = jnp.sum(hi_v)

    for k in range(HV):
        pacc[pl.ds(k * L, L)] = zer
        pacc[pl.ds(H + k * L, L)] = neg
        pacc[pl.ds(2 * H + k * L, L)] = zer
        pacc[pl.ds(3 * H + k * L, L)] = neg

    ybufs = (ybuf0, ybuf1)
    sems = (semy0, semy1)

    def start_y(ci, b):
        pltpu.async_copy(y_hbm.at[pl.ds((lo + ci * CR) * H, CR * H)],
                         ybufs[b], sems[b])

    def wait_y(b):
        pltpu.make_async_copy(y_hbm.at[pl.ds(0, CR * H)],
                              ybufs[b], sems[b]).wait()

    def proc(b, base):
        def rr(j, _):
            r = base + j

            @pl.when(r < hi)
            def _():
                off = jnp.where(r < mid, 0, 2 * H)
                for k in range(HV):
                    v = ybufs[b][pl.ds(j * H + k * L, L)]
                    plsc.addupdate(pacc.at[pl.ds(off + k * L, L)], v)
                    cur = pacc[pl.ds(off + H + k * L, L)]
                    pacc[pl.ds(off + H + k * L, L)] = jnp.maximum(cur, v)
            return 0
        lax.fori_loop(0, CR, rr, 0)

    nchr = (hi - lo + CR - 1) // CR
    nchr = nchr + lax.rem(nchr, 2)
    nchp = nchr // 2

    @pl.when(nchp > 0)
    def _():
        start_y(0, 0)

    def rb(cp, _):
        start_y(2 * cp + 1, 1)
        wait_y(0)
        proc(0, lo + 2 * cp * CR)

        @pl.when(cp + 1 < nchp)
        def _():
            start_y(2 * cp + 2, 0)
        wait_y(1)
        proc(1, lo + (2 * cp + 1) * CR)
        return 0
    lax.fori_loop(0, nchp, rb, 0)

    c0v = jnp.full((L,), mid - lo, jnp.int32)
    c1v = jnp.full((L,), hi - mid, jnp.int32)
    inv0 = 1.0 / jnp.maximum(c0v.astype(jnp.float32), 1.0)
    inv1 = 1.0 / jnp.maximum(c1v.astype(jnp.float32), 1.0)
    ok0 = c0v > 0
    ok1 = c1v > 0
    for k in range(HV):
        zbuf[pl.ds(k * L, L)] = pacc[pl.ds(k * L, L)] * inv0
        zbuf[pl.ds(H + k * L, L)] = jnp.where(
            ok0, pacc[pl.ds(H + k * L, L)], 0.0)
        zbuf[pl.ds(2 * H + k * L, L)] = pacc[pl.ds(2 * H + k * L, L)] * inv1
        zbuf[pl.ds(3 * H + k * L, L)] = jnp.where(
            ok1, pacc[pl.ds(3 * H + k * L, L)], 0.0)
    pltpu.sync_copy(zbuf, z_hbm.at[pl.ds(g0 * 2 * H, 4 * H)])


def _pack_words(lo, hi):
    lob = lax.bitcast_convert_type(lo.astype(jnp.bfloat16),
                                   jnp.int16).astype(jnp.int32) & 0xFFFF
    hib = lax.bitcast_convert_type(hi.astype(jnp.bfloat16),
                                   jnp.int16).astype(jnp.int32)
    return lob | lax.shift_left(hib, 16)


def _t10_body(x_ref, deg_ref, w_ref, wlo_ref, whi_ref, o_ref, ow_ref):
    dinv = lax.rsqrt(deg_ref[...] + 1.0)
    x = x_ref[...]
    o_ref[...] = dinv * jnp.dot(x, w_ref[...],
                                preferred_element_type=jnp.float32)
    lo = dinv * jnp.dot(x, wlo_ref[...], preferred_element_type=jnp.float32)
    hi = dinv * jnp.dot(x, whi_ref[...], preferred_element_type=jnp.float32)
    ow_ref[...] = _pack_words(lo, hi)


def _t1i_body(a_ref, st_ref, g_ref, be_ref, deg_ref, w_ref, wlo_ref,
              whi_ref, o_ref, ow_ref):
    s = st_ref[...]
    mean = s[0:1, :] * (1.0 / N)
    var = s[1:2, :] * (1.0 / N) - mean * mean
    rstd = lax.rsqrt(var + 1e-5)
    y = jnp.maximum((a_ref[...] - mean) * rstd * g_ref[...] + be_ref[...], 0.0)
    dinv = lax.rsqrt(deg_ref[...] + 1.0)
    o_ref[...] = dinv * jnp.dot(y, w_ref[...],
                                preferred_element_type=jnp.float32)
    lo = dinv * jnp.dot(y, wlo_ref[...], preferred_element_type=jnp.float32)
    hi = dinv * jnp.dot(y, whi_ref[...], preferred_element_type=jnp.float32)
    ow_ref[...] = _pack_words(lo, hi)


def _t3_body(s_ref, hp_ref, deg_ref, b_ref, a_ref, st_ref):
    i = pl.program_id(0)
    dinv = lax.rsqrt(deg_ref[...] + 1.0)
    a = dinv * (s_ref[...] + hp_ref[...]) + b_ref[...]
    a_ref[...] = a
    rows = i * BR + lax.broadcasted_iota(jnp.int32, (BR, 1), 0)
    am = jnp.where(rows < N, a, 0.0)
    ps = jnp.concatenate([jnp.sum(am, 0, keepdims=True),
                          jnp.sum(am * am, 0, keepdims=True)], 0)

    @pl.when(i == 0)
    def _():
        st_ref[...] = jnp.zeros_like(st_ref)
    st_ref[...] += ps


def _t4a_body(a_ref, st_ref, g_ref, be_ref, y_ref):
    s = st_ref[...]
    mean = s[0:1, :] * (1.0 / N)
    var = s[1:2, :] * (1.0 / N) - mean * mean
    rstd = lax.rsqrt(var + 1e-5)
    y_ref[...] = jnp.maximum(
        (a_ref[...] - mean) * rstd * g_ref[...] + be_ref[...], 0.0)


def _clf_body(z_ref, w0_ref, b0_ref, w1_ref, b1_ref, w2_ref, b2_ref, o_ref):
    z = jnp.maximum(jnp.dot(z_ref[...], w0_ref[...],
                            preferred_element_type=jnp.float32) + b0_ref[...],
                    0.0)
    z = jnp.maximum(jnp.dot(z, w1_ref[...],
                            preferred_element_type=jnp.float32) + b1_ref[...],
                    0.0)
    lg = jnp.dot(z, w2_ref[...],
                 preferred_element_type=jnp.float32) + b2_ref[...]
    mx = jnp.max(lg, axis=1, keepdims=True)
    sh = lg - mx
    lse = jnp.log(jnp.sum(jnp.exp(sh), axis=1, keepdims=True))
    o_ref[...] = sh - lse


def _row_specs(feat):
    return pl.BlockSpec((BR, feat), lambda i: (i, 0))


def _const_spec(shape):
    return pl.BlockSpec(shape, lambda i: (0, 0))


def _t10(xp, deg_col, W0, Wlo, Whi):
    return pl.pallas_call(
        _t10_body,
        grid=(NPAD // BR,),
        in_specs=[_row_specs(D_IN), _row_specs(1), _const_spec((D_IN, H)),
                  _const_spec((D_IN, H // 2)), _const_spec((D_IN, H // 2))],
        out_specs=[_row_specs(H), _row_specs(H // 2)],
        out_shape=[jax.ShapeDtypeStruct((NPAD, H), jnp.float32),
                   jax.ShapeDtypeStruct((NPAD, H // 2), jnp.int32)],
    )(xp, deg_col, W0, Wlo, Whi)


def _t1i(a, st, g, be, deg_col, W, Wlo, Whi):
    return pl.pallas_call(
        _t1i_body,
        grid=(NPAD // BR,),
        in_specs=[_row_specs(H), _const_spec((2, H)), _const_spec((1, H)),
                  _const_spec((1, H)), _row_specs(1), _const_spec((H, H)),
                  _const_spec((H, H // 2)), _const_spec((H, H // 2))],
        out_specs=[_row_specs(H), _row_specs(H // 2)],
        out_shape=[jax.ShapeDtypeStruct((NPAD, H), jnp.float32),
                   jax.ShapeDtypeStruct((NPAD, H // 2), jnp.int32)],
    )(a, st, g, be, deg_col, W, Wlo, Whi)


def _t3(S, hp, deg_col, b):
    return pl.pallas_call(
        _t3_body,
        grid=(NPAD // BR,),
        in_specs=[_row_specs(H), _row_specs(H), _row_specs(1),
                  _const_spec((1, H))],
        out_specs=[_row_specs(H), _const_spec((2, H))],
        out_shape=[jax.ShapeDtypeStruct((NPAD, H), jnp.float32),
                   jax.ShapeDtypeStruct((2, H), jnp.float32)],
    )(S, hp, deg_col, b)


def _t4a(a, st, g, be):
    return pl.pallas_call(
        _t4a_body,
        grid=(NPAD // BR,),
        in_specs=[_row_specs(H), _const_spec((2, H)), _const_spec((1, H)),
                  _const_spec((1, H))],
        out_specs=_row_specs(H),
        out_shape=jax.ShapeDtypeStruct((NPAD, H), jnp.float32),
    )(a, st, g, be)


def _clf(z, cW0, cb0, cW1, cb1, cW2, cb2):
    return pl.pallas_call(
        _clf_body,
        out_shape=jax.ShapeDtypeStruct((G, C), jnp.float32),
    )(z, cW0, cb0, cW1, cb1, cW2, cb2)


def kernel(x, edge_index, batch, W0, b0, g0, be0, W1, b1, g1, be1,
           W2, b2, g2, be2, cW0, cb0, cW1, cb1, cW2, cb2):
    src = edge_index[0]
    dst = edge_index[1]
    xp = jnp.pad(x, ((0, NPAD - N), (0, 0)))
    batch_p = jnp.pad(batch, (0, NPAD - N), constant_values=127)
    mesh = plsc.VectorSubcoreMesh(core_axis_name="c", subcore_axis_name="s",
                                  num_cores=NC, num_subcores=NS)

    deg, ssrc, offs = pl.kernel(
        _compact_body,
        out_type=(jax.ShapeDtypeStruct((NPAD,), jnp.float32),
                  jax.ShapeDtypeStruct((NW, EMAX), jnp.int32),
                  jax.ShapeDtypeStruct((NW, RW + L), jnp.int32)),
        mesh=mesh,
        compiler_params=_SC_PARAMS,
        scratch_types=[pltpu.VMEM((SCH,), jnp.int32),
                       pltpu.VMEM((SCH,), jnp.int32),
                       pltpu.VMEM((SCH,), jnp.int32),
                       pltpu.VMEM((SCH,), jnp.int32),
                       pltpu.VMEM((L * LCAP,), jnp.int32),
                       pltpu.VMEM((L * LCAP,), jnp.int32),
                       pltpu.VMEM((EMAX,), jnp.int32),
                       pltpu.VMEM((EMAX,), jnp.int32),
                       pltpu.VMEM((EMAX,), jnp.int32),
                       pltpu.VMEM((RW + L,), jnp.float32),
                       pltpu.VMEM((RW + L,), jnp.int32),
                       pltpu.VMEM((RW + L,), jnp.int32),
                       pltpu.SemaphoreType.DMA,
                       pltpu.SemaphoreType.DMA],
    )(src, dst)
    deg_col = deg.reshape(NPAD, 1)

    def _scatter(hp):
        s_flat = pl.kernel(
            _scatter_body,
            out_type=jax.ShapeDtypeStruct((NPAD * H,), jnp.float32),
            mesh=mesh,
            compiler_params=_SC_PARAMS,
            scratch_types=[pltpu.VMEM((CHS,), jnp.int32),
                           pltpu.VMEM((CHS,), jnp.int32),
                           pltpu.VMEM((CHS, H // 2), jnp.int32),
                           pltpu.VMEM((CHS, H // 2), jnp.int32),
                           pltpu.VMEM((RW + L,), jnp.int32),
                           pltpu.VMEM((RW * H,), jnp.float32),
                           pltpu.SemaphoreType.DMA,
                           pltpu.SemaphoreType.DMA,
                           pltpu.SemaphoreType.DMA,
                           pltpu.SemaphoreType.DMA],
        )(hp, ssrc, offs)
        return s_flat.reshape(NPAD, H)

    locols = jnp.asarray([32 * (m // 16) + m % 16 for m in range(H // 2)],
                         dtype=jnp.int32)
    hicols = locols + 16

    def _wsplit(W):
        return jnp.take(W, locols, axis=1), jnp.take(W, hicols, axis=1)

    hp, hpw = _t10(xp, deg_col, W0, *_wsplit(W0))
    layer = [(b0, g0, be0, W1), (b1, g1, be1, W2), (b2, g2, be2, None)]
    a = st = None
    for li, (b, g, be, Wn) in enumerate(layer):
        S = _scatter(hpw)
        a, st = _t3(S, hp, deg_col, b.reshape(1, H))
        if Wn is not None:
            hp, hpw = _t1i(a, st, g.reshape(1, H), be.reshape(1, H),
                           deg_col, Wn, *_wsplit(Wn))

    y = _t4a(a, st, g2.reshape(1, H), be2.reshape(1, H))

    z = pl.kernel(
        _pool_body,
        out_type=jax.ShapeDtypeStruct((G * 2 * H,), jnp.float32),
        mesh=mesh,
        compiler_params=_SC_PARAMS,
        scratch_types=[pltpu.VMEM((SCH,), jnp.int32),
                       pltpu.VMEM((CR * H,), jnp.float32),
                       pltpu.VMEM((CR * H,), jnp.float32),
                       pltpu.VMEM((4 * H,), jnp.float32),
                       pltpu.VMEM((4 * H,), jnp.float32),
                       pltpu.SemaphoreType.DMA,
                       pltpu.SemaphoreType.DMA],
    )(y.reshape(NPAD * H), batch_p).reshape(G, 2 * H)

    return _clf(z, cW0, cb0.reshape(1, H), cW1, cb1.reshape(1, H // 2),
                cW2, cb2.reshape(1, C))


# speedup vs baseline: 1.0926x; 1.0797x over previous
import jax
import jax.numpy as jnp
from jax import lax
from jax.experimental import pallas as pl
from jax.experimental.pallas import tpu as pltpu
from jax.experimental.pallas import tpu_sc as plsc

N = 10000
E = 320000
D_IN = 128
H = 256
C = 230
G = 64

NC = 2
NS = 16
L = 16
NW = NC * NS
RW = 320
NPAD = NW * RW
EMAX = 16384
CH = 64
CHS = 128
SCH = 2000
CR = 16
HV = H // L
BR = 512

_SC_PARAMS = pltpu.CompilerParams(needs_layout_passes=False)


LCAP = 1024


def _compact_body(src_hbm, dst_hbm, deg_hbm, ssrc_hbm, offs_hbm,
                  sbuf0, sbuf1, dbuf0, dbuf1, lsrc, ldst, csrc, cdst,
                  ssrc, degacc, offbuf, curoff, sems, semd):
    w = lax.axis_index("c") * NS + lax.axis_index("s")
    lo = w * RW
    lanes = lax.iota(jnp.int32, L)
    zer = jnp.zeros((L,), jnp.float32)

    def zb(i, _):
        degacc[pl.ds(i * L, L)] = zer
        return 0
    lax.fori_loop(0, (RW + L) // L, zb, 0)

    lanebase = lanes * LCAP
    NSC = E // SCH
    sbufs = (sbuf0, sbuf1)
    dbufs = (dbuf0, dbuf1)

    def start(ci, b):
        pltpu.async_copy(src_hbm.at[pl.ds(ci * SCH, SCH)], sbufs[b], sems)
        pltpu.async_copy(dst_hbm.at[pl.ds(ci * SCH, SCH)], dbufs[b], semd)

    def wait(b):
        pltpu.make_async_copy(src_hbm.at[pl.ds(0, SCH)],
                              sbufs[b], sems).wait()
        pltpu.make_async_copy(dst_hbm.at[pl.ds(0, SCH)],
                              dbufs[b], semd).wait()

    def scan(b, ptrv):
        def vec(i, ptrv):
            d16 = dbufs[b][pl.ds(i * L, L)]
            s16 = sbufs[b][pl.ds(i * L, L)]
            dl = d16 - lo
            m = (dl >= 0) & (dl < RW)
            pos = lanebase + jnp.minimum(ptrv, LCAP - 1)
            plsc.store_scatter(ldst, [pos], dl, mask=m)
            plsc.store_scatter(lsrc, [pos], s16, mask=m)
            return ptrv + m.astype(jnp.int32)
        return lax.fori_loop(0, SCH // L, vec, ptrv)

    start(0, 0)

    def chunk(cp, ptrv):
        start(2 * cp + 1, 1)
        wait(0)
        ptrv = scan(0, ptrv)

        @pl.when(cp + 1 < NSC // 2)
        def _():
            start(2 * cp + 2, 0)
        wait(1)
        return scan(1, ptrv)
    ptrv = lax.fori_loop(0, NSC // 2, chunk, jnp.zeros((L,), jnp.int32))

    def merge(l, ptr):
        cl = jnp.minimum(ptrv[l], LCAP)
        nv = (cl + L - 1) // L

        def mv(i, _):
            cdst[pl.ds(ptr + i * L, L)] = ldst[pl.ds(l * LCAP + i * L, L)]
            csrc[pl.ds(ptr + i * L, L)] = lsrc[pl.ds(l * LCAP + i * L, L)]
            return 0
        lax.fori_loop(0, nv, mv, 0)
        return jnp.minimum(ptr + cl, EMAX - 2 * CHS - L)
    ptr = jnp.int32(0)
    for l in range(L):
        ptr = merge(l, ptr)

    dum_d = jnp.full((L,), RW, jnp.int32)
    dum_s = jnp.zeros((L,), jnp.int32)
    for k in range(2 * CHS // L):
        cdst[pl.ds(ptr + k * L, L)] = dum_d
        csrc[pl.ds(ptr + k * L, L)] = dum_s

    ones = jnp.ones((L,), jnp.float32)
    nvec = (ptr + L - 1) // L

    def degb(i, _):
        dl = cdst[pl.ds(i * L, L)]
        plsc.addupdate_scatter(degacc, [dl], ones)
        return 0
    lax.fori_loop(0, nvec, degb, 0)

    carry = jnp.int32(0)
    for v in range(RW // L):
        c16 = degacc[pl.ds(v * L, L)].astype(jnp.int32)
        incl = plsc.cumsum(c16)
        offbuf[pl.ds(v * L, L)] = incl - c16 + carry
        carry = carry + jnp.max(incl)
    offbuf[pl.ds(RW, L)] = jnp.full((L,), carry, jnp.int32)

    def cpb(i, _):
        curoff[pl.ds(i * L, L)] = offbuf[pl.ds(i * L, L)]
        return 0
    lax.fori_loop(0, (RW + L) // L, cpb, 0)

    def place(i, _):
        dl = cdst[pl.ds(i * L, L)]
        s16 = csrc[pl.ds(i * L, L)]
        for l in range(L):
            ml = lanes == l
            pos = plsc.load_gather(curoff, [dl], mask=ml)
            plsc.store_scatter(ssrc, [pos], s16, mask=ml)
            plsc.store_scatter(curoff, [dl], pos + 1, mask=ml)
        return 0
    lax.fori_loop(0, nvec, place, 0)

    zs = jnp.zeros((L,), jnp.int32)
    for k in range(2 * CHS // L):
        ssrc[pl.ds(carry + k * L, L)] = zs

    pltpu.sync_copy(degacc.at[pl.ds(0, RW)], deg_hbm.at[pl.ds(lo, RW)])
    pltpu.sync_copy(ssrc, ssrc_hbm.at[w])
    pltpu.sync_copy(offbuf, offs_hbm.at[w])


def _scatter_body(hp_hbm, ssrc_hbm, offs_hbm, s_hbm,
                  idxv0, idxv1, rows0, rows1, offbuf, acc,
                  gsem0, gsem1, isem0, isem1):
    w = lax.axis_index("c") * NS + lax.axis_index("s")
    pltpu.sync_copy(offs_hbm.at[w], offbuf)
    total = offbuf[pl.ds(RW, L)][0]
    nch = (total + CHS - 1) // CHS
    nch = jnp.maximum(nch + lax.rem(nch, 2), 2)
    zer = jnp.zeros((L,), jnp.float32)

    def zb(i, _):
        for k in range(HV):
            acc[i, pl.ds(k * L, L)] = zer
        return 0
    lax.fori_loop(0, RW, zb, 0)

    idxvs = (idxv0, idxv1)
    rowss = (rows0, rows1)
    isems = (isem0, isem1)
    gsems = (gsem0, gsem1)

    def start_idx(ci, b):
        pltpu.async_copy(ssrc_hbm.at[w, pl.ds(ci * CHS, CHS)],
                         idxvs[b], isems[b])

    def wait_idx(b):
        pltpu.make_async_copy(ssrc_hbm.at[w, pl.ds(0, CHS)],
                              idxvs[b], isems[b]).wait()

    def start_gather(b):
        pltpu.async_copy(hp_hbm.at[idxvs[b]], rowss[b], gsems[b])

    def wait_gather(b):
        pltpu.make_async_copy(hp_hbm.at[pl.ds(0, CHS)],
                              rowss[b], gsems[b]).wait()

    def accum(b, base, r):
        rowsb = rowss[b]
        hi_chunk = jnp.minimum(base + CHS, total)

        def cond(st):
            r, e = st
            return (r < RW) & (e < hi_chunk)

        def body(st):
            r, e = st
            off_r = offbuf[pl.ds(r, L)][0]
            off_r1 = offbuf[pl.ds(r + 1, L)][0]
            hi_e = jnp.minimum(off_r1, hi_chunk)
            cmask = jnp.full((L,), e > off_r)
            regs = tuple(
                jnp.where(cmask, acc[r, pl.ds(k * L, L)], 0.0)
                for k in range(HV))

            def ebody(e, regs):
                eb = e - base
                out = []
                for k in range(H // 32):
                    v32i = rowsb[eb, pl.ds(k * L, L)]
                    v32 = plsc.bitcast(v32i, jnp.bfloat16)
                    va, vb = plsc.unpack(
                        v32, format=plsc.PackFormat.INTERLEAVED,
                        preferred_element_type=jnp.float32)
                    out.append(regs[2 * k] + va)
                    out.append(regs[2 * k + 1] + vb)
                return tuple(out)
            regs = lax.fori_loop(e, hi_e, ebody, regs)
            for k in range(HV):
                acc[r, pl.ds(k * L, L)] = regs[k]
            r = jnp.where(off_r1 <= hi_chunk, r + 1, r)
            return (r, hi_e)
        r, _ = lax.while_loop(cond, body, (r, base))
        return r

    start_idx(0, 0)
    start_idx(1, 1)
    wait_idx(0)
    start_gather(0)
    nchp = nch // 2

    def cb(cp, r):
        wait_idx(1)
        start_gather(1)
        wait_gather(0)
        r = accum(0, 2 * cp * CHS, r)

        @pl.when(cp + 1 < nchp)
        def _():
            start_idx(2 * cp + 2, 0)
            wait_idx(0)
            start_gather(0)
        wait_gather(1)
        r = accum(1, (2 * cp + 1) * CHS, r)

        @pl.when(cp + 1 < nchp)
        def _():
            start_idx(2 * cp + 3, 1)
        return r
    lax.fori_loop(0, nchp, cb, jnp.int32(0))
    pltpu.sync_copy(acc, s_hbm.at[pl.ds(w * RW, RW)])


def _pool_body(y_hbm, batch_hbm, z_hbm, bbuf, ybuf0, ybuf1, zbuf, pacc,
               semy0, semy1):
    w = lax.axis_index("c") * NS + lax.axis_index("s")
    g0 = 2 * w
    zer = jnp.zeros((L,), jnp.float32)
    neg = jnp.full((L,), -3.0e38, jnp.float32)

    def cb(ci, carry):
        pltpu.sync_copy(batch_hbm.at[pl.ds(ci * SCH, SCH)], bbuf)

        def vb(i, carry):
            lo_v, mid_v, hi_v = carry
            b16 = bbuf[pl.ds(i * L, L)]
            lo_v = lo_v + (b16 < g0).astype(jnp.int32)
            mid_v = mid_v + (b16 < g0 + 1).astype(jnp.int32)
            hi_v = hi_v + (b16 < g0 + 2).astype(jnp.int32)
            return (lo_v, mid_v, hi_v)
        return lax.fori_loop(0, SCH // L, vb, carry)
    z0v = jnp.zeros((L,), jnp.int32)
    lo_v, mid_v, hi_v = lax.fori_loop(0, NPAD // SCH, cb, (z0v, z0v, z0v))
    lo = jnp.sum(lo_v)
    mid = jnp.sum(mid_v)
    hi = jnp.sum(hi_v)

    for k in range(HV):
        pacc[pl.ds(k * L, L)] = zer
        pacc[pl.ds(H + k * L, L)] = neg
        pacc[pl.ds(2 * H + k * L, L)] = zer
        pacc[pl.ds(3 * H + k * L, L)] = neg

    ybufs = (ybuf0, ybuf1)
    sems = (semy0, semy1)

    def start_y(ci, b):
        pltpu.async_copy(y_hbm.at[pl.ds((lo + ci * CR) * H, CR * H)],
                         ybufs[b], sems[b])

    def wait_y(b):
        pltpu.make_async_copy(y_hbm.at[pl.ds(0, CR * H)],
                              ybufs[b], sems[b]).wait()

    def proc(b, base):
        def rr(j, _):
            r = base + j

            @pl.when(r < hi)
            def _():
                off = jnp.where(r < mid, 0, 2 * H)
                for k in range(HV):
                    v = ybufs[b][pl.ds(j * H + k * L, L)]
                    plsc.addupdate(pacc.at[pl.ds(off + k * L, L)], v)
                    cur = pacc[pl.ds(off + H + k * L, L)]
                    pacc[pl.ds(off + H + k * L, L)] = jnp.maximum(cur, v)
            return 0
        lax.fori_loop(0, CR, rr, 0)

    nchr = (hi - lo + CR - 1) // CR
    nchr = nchr + lax.rem(nchr, 2)
    nchp = nchr // 2

    @pl.when(nchp > 0)
    def _():
        start_y(0, 0)

    def rb(cp, _):
        start_y(2 * cp + 1, 1)
        wait_y(0)
        proc(0, lo + 2 * cp * CR)

        @pl.when(cp + 1 < nchp)
        def _():
            start_y(2 * cp + 2, 0)
        wait_y(1)
        proc(1, lo + (2 * cp + 1) * CR)
        return 0
    lax.fori_loop(0, nchp, rb, 0)

    c0v = jnp.full((L,), mid - lo, jnp.int32)
    c1v = jnp.full((L,), hi - mid, jnp.int32)
    inv0 = 1.0 / jnp.maximum(c0v.astype(jnp.float32), 1.0)
    inv1 = 1.0 / jnp.maximum(c1v.astype(jnp.float32), 1.0)
    ok0 = c0v > 0
    ok1 = c1v > 0
    for k in range(HV):
        zbuf[pl.ds(k * L, L)] = pacc[pl.ds(k * L, L)] * inv0
        zbuf[pl.ds(H + k * L, L)] = jnp.where(
            ok0, pacc[pl.ds(H + k * L, L)], 0.0)
        zbuf[pl.ds(2 * H + k * L, L)] = pacc[pl.ds(2 * H + k * L, L)] * inv1
        zbuf[pl.ds(3 * H + k * L, L)] = jnp.where(
            ok1, pacc[pl.ds(3 * H + k * L, L)], 0.0)
    pltpu.sync_copy(zbuf, z_hbm.at[pl.ds(g0 * 2 * H, 4 * H)])


def _pack_words(lo, hi):
    lob = lax.bitcast_convert_type(lo.astype(jnp.bfloat16),
                                   jnp.int16).astype(jnp.int32) & 0xFFFF
    hib = lax.bitcast_convert_type(hi.astype(jnp.bfloat16),
                                   jnp.int16).astype(jnp.int32)
    return lob | lax.shift_left(hib, 16)


def _t10_body(x_ref, deg_ref, w_ref, wlo_ref, whi_ref, o_ref, ow_ref):
    dinv = lax.rsqrt(deg_ref[...] + 1.0)
    x = x_ref[...]
    o_ref[...] = dinv * jnp.dot(x, w_ref[...],
                                preferred_element_type=jnp.float32)
    lo = dinv * jnp.dot(x, wlo_ref[...], preferred_element_type=jnp.float32)
    hi = dinv * jnp.dot(x, whi_ref[...], preferred_element_type=jnp.float32)
    ow_ref[...] = _pack_words(lo, hi)


def _t1i_body(a_ref, st_ref, g_ref, be_ref, deg_ref, w_ref, wlo_ref,
              whi_ref, o_ref, ow_ref):
    s = st_ref[...]
    mean = s[0:1, :] * (1.0 / N)
    var = s[1:2, :] * (1.0 / N) - mean * mean
    rstd = lax.rsqrt(var + 1e-5)
    y = jnp.maximum((a_ref[...] - mean) * rstd * g_ref[...] + be_ref[...], 0.0)
    dinv = lax.rsqrt(deg_ref[...] + 1.0)
    o_ref[...] = dinv * jnp.dot(y, w_ref[...],
                                preferred_element_type=jnp.float32)
    lo = dinv * jnp.dot(y, wlo_ref[...], preferred_element_type=jnp.float32)
    hi = dinv * jnp.dot(y, whi_ref[...], preferred_element_type=jnp.float32)
    ow_ref[...] = _pack_words(lo, hi)


def _t3_body(s_ref, hp_ref, deg_ref, b_ref, a_ref, st_ref):
    i = pl.program_id(0)
    dinv = lax.rsqrt(deg_ref[...] + 1.0)
    a = dinv * (s_ref[...] + hp_ref[...]) + b_ref[...]
    a_ref[...] = a
    rows = i * BR + lax.broadcasted_iota(jnp.int32, (BR, 1), 0)
    am = jnp.where(rows < N, a, 0.0)
    ps = jnp.concatenate([jnp.sum(am, 0, keepdims=True),
                          jnp.sum(am * am, 0, keepdims=True)], 0)

    @pl.when(i == 0)
    def _():
        st_ref[...] = jnp.zeros_like(st_ref)
    st_ref[...] += ps


def _t4a_body(a_ref, st_ref, g_ref, be_ref, y_ref):
    s = st_ref[...]
    mean = s[0:1, :] * (1.0 / N)
    var = s[1:2, :] * (1.0 / N) - mean * mean
    rstd = lax.rsqrt(var + 1e-5)
    y_ref[...] = jnp.maximum(
        (a_ref[...] - mean) * rstd * g_ref[...] + be_ref[...], 0.0)


def _clf_body(z_ref, w0_ref, b0_ref, w1_ref, b1_ref, w2_ref, b2_ref, o_ref):
    z = jnp.maximum(jnp.dot(z_ref[...], w0_ref[...],
                            preferred_element_type=jnp.float32) + b0_ref[...],
                    0.0)
    z = jnp.maximum(jnp.dot(z, w1_ref[...],
                            preferred_element_type=jnp.float32) + b1_ref[...],
                    0.0)
    lg = jnp.dot(z, w2_ref[...],
                 preferred_element_type=jnp.float32) + b2_ref[...]
    mx = jnp.max(lg, axis=1, keepdims=True)
    sh = lg - mx
    lse = jnp.log(jnp.sum(jnp.exp(sh), axis=1, keepdims=True))
    o_ref[...] = sh - lse


def _row_specs(feat):
    return pl.BlockSpec((BR, feat), lambda i: (i, 0))


def _const_spec(shape):
    return pl.BlockSpec(shape, lambda i: (0, 0))


def _t10(xp, deg_col, W0, Wlo, Whi):
    return pl.pallas_call(
        _t10_body,
        grid=(NPAD // BR,),
        in_specs=[_row_specs(D_IN), _row_specs(1), _const_spec((D_IN, H)),
                  _const_spec((D_IN, H // 2)), _const_spec((D_IN, H // 2))],
        out_specs=[_row_specs(H), _row_specs(H // 2)],
        out_shape=[jax.ShapeDtypeStruct((NPAD, H), jnp.float32),
                   jax.ShapeDtypeStruct((NPAD, H // 2), jnp.int32)],
    )(xp, deg_col, W0, Wlo, Whi)


def _t1i(a, st, g, be, deg_col, W, Wlo, Whi):
    return pl.pallas_call(
        _t1i_body,
        grid=(NPAD // BR,),
        in_specs=[_row_specs(H), _const_spec((2, H)), _const_spec((1, H)),
                  _const_spec((1, H)), _row_specs(1), _const_spec((H, H)),
                  _const_spec((H, H // 2)), _const_spec((H, H // 2))],
        out_specs=[_row_specs(H), _row_specs(H // 2)],
        out_shape=[jax.ShapeDtypeStruct((NPAD, H), jnp.float32),
                   jax.ShapeDtypeStruct((NPAD, H // 2), jnp.int32)],
    )(a, st, g, be, deg_col, W, Wlo, Whi)


def _t3(S, hp, deg_col, b):
    return pl.pallas_call(
        _t3_body,
        grid=(NPAD // BR,),
        in_specs=[_row_specs(H), _row_specs(H), _row_specs(1),
                  _const_spec((1, H))],
        out_specs=[_row_specs(H), _const_spec((2, H))],
        out_shape=[jax.ShapeDtypeStruct((NPAD, H), jnp.float32),
                   jax.ShapeDtypeStruct((2, H), jnp.float32)],
    )(S, hp, deg_col, b)


def _t4a(a, st, g, be):
    return pl.pallas_call(
        _t4a_body,
        grid=(NPAD // BR,),
        in_specs=[_row_specs(H), _const_spec((2, H)), _const_spec((1, H)),
                  _const_spec((1, H))],
        out_specs=_row_specs(H),
        out_shape=jax.ShapeDtypeStruct((NPAD, H), jnp.float32),
    )(a, st, g, be)


def _clf(z, cW0, cb0, cW1, cb1, cW2, cb2):
    return pl.pallas_call(
        _clf_body,
        out_shape=jax.ShapeDtypeStruct((G, C), jnp.float32),
    )(z, cW0, cb0, cW1, cb1, cW2, cb2)


def kernel(x, edge_index, batch, W0, b0, g0, be0, W1, b1, g1, be1,
           W2, b2, g2, be2, cW0, cb0, cW1, cb1, cW2, cb2):
    src = edge_index[0]
    dst = edge_index[1]
    xp = jnp.pad(x, ((0, NPAD - N), (0, 0)))
    batch_p = jnp.pad(batch, (0, NPAD - N), constant_values=127)
    mesh = plsc.VectorSubcoreMesh(core_axis_name="c", subcore_axis_name="s",
                                  num_cores=NC, num_subcores=NS)

    deg, ssrc, offs = pl.kernel(
        _compact_body,
        out_type=(jax.ShapeDtypeStruct((NPAD,), jnp.float32),
                  jax.ShapeDtypeStruct((NW, EMAX), jnp.int32),
                  jax.ShapeDtypeStruct((NW, RW + L), jnp.int32)),
        mesh=mesh,
        compiler_params=_SC_PARAMS,
        scratch_types=[pltpu.VMEM((SCH,), jnp.int32),
                       pltpu.VMEM((SCH,), jnp.int32),
                       pltpu.VMEM((SCH,), jnp.int32),
                       pltpu.VMEM((SCH,), jnp.int32),
                       pltpu.VMEM((L * LCAP,), jnp.int32),
                       pltpu.VMEM((L * LCAP,), jnp.int32),
                       pltpu.VMEM((EMAX,), jnp.int32),
                       pltpu.VMEM((EMAX,), jnp.int32),
                       pltpu.VMEM((EMAX,), jnp.int32),
                       pltpu.VMEM((RW + L,), jnp.float32),
                       pltpu.VMEM((RW + L,), jnp.int32),
                       pltpu.VMEM((RW + L,), jnp.int32),
                       pltpu.SemaphoreType.DMA,
                       pltpu.SemaphoreType.DMA],
    )(src, dst)
    deg_col = deg.reshape(NPAD, 1)

    def _scatter(hp):
        s_flat = pl.kernel(
            _scatter_body,
            out_type=jax.ShapeDtypeStruct((NPAD, H), jnp.float32),
            mesh=mesh,
            compiler_params=_SC_PARAMS,
            scratch_types=[pltpu.VMEM((CHS,), jnp.int32),
                           pltpu.VMEM((CHS,), jnp.int32),
                           pltpu.VMEM((CHS, H // 2), jnp.int32),
                           pltpu.VMEM((CHS, H // 2), jnp.int32),
                           pltpu.VMEM((RW + L,), jnp.int32),
                           pltpu.VMEM((RW, H), jnp.float32),
                           pltpu.SemaphoreType.DMA,
                           pltpu.SemaphoreType.DMA,
                           pltpu.SemaphoreType.DMA,
                           pltpu.SemaphoreType.DMA],
        )(hp, ssrc, offs)
        return s_flat

    locols = jnp.asarray([32 * (m // 16) + m % 16 for m in range(H // 2)],
                         dtype=jnp.int32)
    hicols = locols + 16

    def _wsplit(W):
        return jnp.take(W, locols, axis=1), jnp.take(W, hicols, axis=1)

    hp, hpw = _t10(xp, deg_col, W0, *_wsplit(W0))
    layer = [(b0, g0, be0, W1), (b1, g1, be1, W2), (b2, g2, be2, None)]
    a = st = None
    for li, (b, g, be, Wn) in enumerate(layer):
        S = _scatter(hpw)
        a, st = _t3(S, hp, deg_col, b.reshape(1, H))
        if Wn is not None:
            hp, hpw = _t1i(a, st, g.reshape(1, H), be.reshape(1, H),
                           deg_col, Wn, *_wsplit(Wn))

    y = _t4a(a, st, g2.reshape(1, H), be2.reshape(1, H))

    z = pl.kernel(
        _pool_body,
        out_type=jax.ShapeDtypeStruct((G * 2 * H,), jnp.float32),
        mesh=mesh,
        compiler_params=_SC_PARAMS,
        scratch_types=[pltpu.VMEM((SCH,), jnp.int32),
                       pltpu.VMEM((CR * H,), jnp.float32),
                       pltpu.VMEM((CR * H,), jnp.float32),
                       pltpu.VMEM((4 * H,), jnp.float32),
                       pltpu.VMEM((4 * H,), jnp.float32),
                       pltpu.SemaphoreType.DMA,
                       pltpu.SemaphoreType.DMA],
    )(y.reshape(NPAD * H), batch_p).reshape(G, 2 * H)

    return _clf(z, cW0, cb0.reshape(1, H), cW1, cb1.reshape(1, H // 2),
                cW2, cb2.reshape(1, C))
